# baseline (speedup 1.0000x reference)
import functools

import jax
import jax.numpy as jnp
from jax import lax
from jax.experimental import pallas as pl
from jax.experimental.pallas import tpu as pltpu
from jax.experimental.pallas import tpu_sc as plsc

N = 10000
E = 320000
NW = 32
CH = 80
CHS = 40
EPW = CH * 128
EPAD = NW * EPW
N1 = 10112
RPT = N1 // 16
DEGW = 16

_MESH = plsc.VectorSubcoreMesh(core_axis_name="c", subcore_axis_name="s")


def _sc_degree():
    @functools.partial(
        pl.kernel,
        out_type=jax.ShapeDtypeStruct((2, N1, DEGW), jnp.float32),
        mesh=_MESH,
        scratch_types=[
            pltpu.VMEM((CH, 128), jnp.int32),
            pltpu.VMEM((128, DEGW), jnp.float32),
            pltpu.VMEM_SHARED((N1, DEGW), jnp.float32),
        ],
        compiler_params=pltpu.CompilerParams(use_tc_tiling_on_sc=False),
    )
    def k(dstp_hbm, ones_hbm, zeros_hbm, out_hbm, dst_v, ones_v, acc):
        c = lax.axis_index("c")
        s = lax.axis_index("s")
        wid = s * 2 + c
        row0 = s * RPT
        pltpu.sync_copy(zeros_hbm.at[pl.ds(row0, RPT)], acc.at[pl.ds(row0, RPT)])
        pltpu.sync_copy(dstp_hbm.at[wid], dst_v)
        pltpu.sync_copy(ones_hbm, ones_v)
        plsc.subcore_barrier()

        def body(j, carry):
            pltpu.sync_copy(ones_v, acc.at[dst_v.at[j]], add=True)
            return carry

        lax.fori_loop(0, CH, body, 0)
        plsc.subcore_barrier()
        pltpu.sync_copy(acc.at[pl.ds(row0, RPT)], out_hbm.at[c, pl.ds(row0, RPT)])

    return k


def _sc_spmm(D):
    @functools.partial(
        pl.kernel,
        out_type=jax.ShapeDtypeStruct((2, N1, D), jnp.float32),
        mesh=_MESH,
        scratch_types=[
            pltpu.VMEM((CHS, 128), jnp.int32),
            pltpu.VMEM((CHS, 128), jnp.int32),
            pltpu.VMEM((2, 128, D), jnp.float32),
            pltpu.VMEM_SHARED((N1, D), jnp.float32),
            pltpu.SemaphoreType.DMA,
            pltpu.SemaphoreType.DMA,
        ],
        compiler_params=pltpu.CompilerParams(use_tc_tiling_on_sc=False),
    )
    def k(g_hbm, srcp_hbm, dstp_hbm, zeros_hbm, out_hbm,
          src_v, dst_v, rows_v, acc, sem0, sem1):
        c = lax.axis_index("c")
        s = lax.axis_index("s")
        wid = s * 2 + c
        row0 = s * RPT
        pltpu.sync_copy(zeros_hbm.at[pl.ds(row0, RPT)], acc.at[pl.ds(row0, RPT)])
        plsc.subcore_barrier()

        sems = (sem0, sem1)
        for half in range(CH // CHS):
            pltpu.sync_copy(srcp_hbm.at[wid, pl.ds(half * CHS, CHS)], src_v)
            pltpu.sync_copy(dstp_hbm.at[wid, pl.ds(half * CHS, CHS)], dst_v)
            pltpu.async_copy(g_hbm.at[src_v.at[0]], rows_v.at[0], sem0)
            pltpu.async_copy(g_hbm.at[src_v.at[1]], rows_v.at[1], sem1)

            def body(i, carry):
                g0 = i * 2
                for b in range(2):
                    j = g0 + b
                    pltpu.make_async_copy(
                        g_hbm.at[src_v.at[j]], rows_v.at[b], sems[b]).wait()
                    pltpu.sync_copy(rows_v.at[b], acc.at[dst_v.at[j]], add=True)

                    @pl.when(j + 2 < CHS)
                    def _():
                        pltpu.async_copy(
                            g_hbm.at[src_v.at[j + 2]], rows_v.at[b], sems[b])
                return carry

            lax.fori_loop(0, CHS // 2, body, 0)
        plsc.subcore_barrier()
        pltpu.sync_copy(acc.at[pl.ds(row0, RPT)], out_hbm.at[c, pl.ds(row0, RPT)])

    return k


def _sc_spmm_deep(D, NBUF=4, G=2):
    @functools.partial(
        pl.kernel,
        out_type=jax.ShapeDtypeStruct((2, N1, D), jnp.float32),
        mesh=_MESH,
        scratch_types=[
            pltpu.VMEM((CH, 128), jnp.int32),
            pltpu.VMEM((CH, 128), jnp.int32),
            pltpu.VMEM((NBUF, 128, D), jnp.float32),
            pltpu.VMEM_SHARED((N1, D), jnp.float32),
            [pltpu.SemaphoreType.DMA] * NBUF,
            [pltpu.SemaphoreType.DMA] * NBUF,
        ],
        compiler_params=pltpu.CompilerParams(
            use_tc_tiling_on_sc=False) if D % 128 else None,
    )
    def k(g_hbm, srcp_hbm, dstp_hbm, zeros_hbm, out_hbm,
          src_v, dst_v, rows_v, acc, gsem, ssem):
        c = lax.axis_index("c")
        s = lax.axis_index("s")
        wid = s * 2 + c
        row0 = s * RPT
        pltpu.sync_copy(zeros_hbm.at[pl.ds(row0, RPT)], acc.at[pl.ds(row0, RPT)])
        pltpu.sync_copy(srcp_hbm.at[wid], src_v)
        pltpu.sync_copy(dstp_hbm.at[wid], dst_v)
        plsc.subcore_barrier()

        for j in range(G):
            pltpu.async_copy(g_hbm.at[src_v.at[j]], rows_v.at[j], gsem[j])

        def body(io, carry):
            j0 = io * NBUF
            for b in range(NBUF):
                j = j0 + b
                pltpu.make_async_copy(
                    g_hbm.at[src_v.at[j]], rows_v.at[b], gsem[b]).wait()
                pltpu.async_copy(rows_v.at[b], acc.at[dst_v.at[j]], ssem[b],
                                 add=True)
                k_next = j + G
                rk = (b + G) % NBUF

                @pl.when(k_next < CH)
                def _():
                    @pl.when(k_next >= NBUF)
                    def _():
                        pltpu.make_async_copy(
                            rows_v.at[rk],
                            acc.at[dst_v.at[jnp.maximum(k_next - NBUF, 0)]],
                            ssem[rk]).wait()

                    pltpu.async_copy(
                        g_hbm.at[src_v.at[k_next]], rows_v.at[rk], gsem[rk])
            return carry

        lax.fori_loop(0, CH // NBUF, body, 0)
        for d in range(NBUF):
            j = CH - NBUF + d
            pltpu.make_async_copy(
                rows_v.at[j % NBUF], acc.at[dst_v.at[j]],
                ssem[j % NBUF]).wait()
        plsc.subcore_barrier()
        pltpu.sync_copy(acc.at[pl.ds(row0, RPT)], out_hbm.at[c, pl.ds(row0, RPT)])

    return k


def _sc_diag(D, gath, scat, NB=2):
    @functools.partial(
        pl.kernel,
        out_type=jax.ShapeDtypeStruct((2, N1, D), jnp.float32),
        mesh=_MESH,
        scratch_types=[
            pltpu.VMEM((CH if gath else 1, 128), jnp.int32),
            pltpu.VMEM((CH if scat else 1, 128), jnp.int32),
            pltpu.VMEM((NB, 128, D), jnp.float32),
            pltpu.VMEM_SHARED((N1 if scat else 16, D), jnp.float32),
            [pltpu.SemaphoreType.DMA] * NB,
            [pltpu.SemaphoreType.DMA] * NB,
        ],
        compiler_params=pltpu.CompilerParams(
            use_tc_tiling_on_sc=False) if D % 128 else None,
    )
    def k(g_hbm, srcp_hbm, dstp_hbm, zeros_hbm, out_hbm,
          src_v, dst_v, rows_v, acc, gsem, ssem):
        c = lax.axis_index("c")
        s = lax.axis_index("s")
        wid = s * 2 + c
        row0 = s * RPT
        if gath:
            pltpu.sync_copy(srcp_hbm.at[wid], src_v)
        if scat:
            pltpu.sync_copy(dstp_hbm.at[wid], dst_v)
        plsc.subcore_barrier()

        if gath:
            for j in range(NB):
                pltpu.async_copy(g_hbm.at[src_v.at[j]], rows_v.at[j], gsem[j])

        def body(io, carry):
            j0 = io * NB
            for b in range(NB):
                j = j0 + b
                if gath:
                    pltpu.make_async_copy(
                        g_hbm.at[src_v.at[j]], rows_v.at[b], gsem[b]).wait()
                if scat:
                    @pl.when(j >= NB)
                    def _():
                        pltpu.make_async_copy(
                            rows_v.at[b], acc.at[dst_v.at[0]], ssem[b]).wait()
                    pltpu.async_copy(rows_v.at[b], acc.at[dst_v.at[j]],
                                     ssem[b], add=True)
                if gath:
                    @pl.when(j + NB < CH)
                    def _():
                        pltpu.async_copy(
                            g_hbm.at[src_v.at[j]], rows_v.at[b], gsem[b])
            return carry

        lax.fori_loop(0, CH // NB, body, 0)
        if scat:
            for d in range(NB):
                pltpu.make_async_copy(
                    rows_v.at[d], acc.at[dst_v.at[0]], ssem[d]).wait()
        plsc.subcore_barrier()
        pltpu.sync_copy(zeros_hbm.at[pl.ds(row0, RPT)],
                        out_hbm.at[c, pl.ds(row0, RPT)])

    return k


_RB = 1000


def _row_spec(D):
    return pl.BlockSpec((_RB, D), lambda i: (i, 0))


def _part_specs(D):
    return [pl.BlockSpec((1, _RB, D), lambda i, c=c: (c, i, 0)) for c in (0, 1)]


def _full_spec(shape):
    return pl.BlockSpec(shape, lambda i: tuple(0 for _ in shape))


def _tc_call(body, in_specs, out_specs, out_shapes, args):
    return pl.pallas_call(
        body,
        grid=(N // _RB,),
        in_specs=in_specs,
        out_specs=out_specs,
        out_shape=out_shapes,
    )(*args)


def _tc_b(degp, x, w1t):
    def body(p0, p1, x_r, w_r, dinv_r, g_r):
        deg = p0[0, :, 0:1] + p1[0, :, 0:1] + 1.0
        dinv = lax.rsqrt(deg)
        dinv_r[...] = dinv
        g_r[...] = dinv * jnp.dot(x_r[...], w_r[...],
                                  preferred_element_type=jnp.float32)

    return _tc_call(
        body,
        _part_specs(DEGW) + [_row_spec(128), _full_spec((128, 128))],
        [_row_spec(1), _row_spec(128)],
        [jax.ShapeDtypeStruct((N, 1), jnp.float32),
         jax.ShapeDtypeStruct((N, 128), jnp.float32)],
        (degp, degp, x, w1t),
    )


def _tc_mid(parts, g, dinv, b, wnext, D, DN):
    def body(p0, p1, g_r, dinv_r, b_r, w_r, gn_r):
        h = dinv_r[...] * (p0[0] + p1[0] + g_r[...]) + b_r[...]
        gn_r[...] = dinv_r[...] * jnp.dot(h, w_r[...],
                                          preferred_element_type=jnp.float32)

    return _tc_call(
        body,
        _part_specs(D) + [_row_spec(D), _row_spec(1), _full_spec((1, D)),
                          _full_spec((D, DN))],
        [_row_spec(DN)],
        [jax.ShapeDtypeStruct((N, DN), jnp.float32)],
        (parts, parts, g, dinv, b, wnext),
    )[0]


def _tc_f(parts, g2, dinv, b2, head1, w2):
    def body(p0, p1, g_r, dinv_r, b_r, h1_r, w2_r, h2_r, z_r, g3_r):
        h2 = dinv_r[...] * (p0[0] + p1[0] + g_r[...]) + b_r[...]
        z = jnp.dot(h2, h1_r[...], preferred_element_type=jnp.float32)
        h2_r[...] = h2
        z_r[...] = z
        g3_r[...] = dinv_r[...] * jnp.dot(z, w2_r[...],
                                          preferred_element_type=jnp.float32)

    return _tc_call(
        body,
        _part_specs(64) + [_row_spec(64), _row_spec(1), _full_spec((1, 64)),
                           _full_spec((64, 64)), _full_spec((64, 128))],
        [_row_spec(64), _row_spec(64), _row_spec(128)],
        [jax.ShapeDtypeStruct((N, 64), jnp.float32),
         jax.ShapeDtypeStruct((N, 64), jnp.float32),
         jax.ShapeDtypeStruct((N, 128), jnp.float32)],
        (parts, parts, g2, dinv, b2, head1, w2),
    )


def _tc_last(parts, g4, dinv, b4):
    def body(p0, p1, g_r, dinv_r, b_r, h_r):
        h_r[...] = dinv_r[...] * (p0[0] + p1[0] + g_r[...]) + b_r[...]

    return _tc_call(
        body,
        _part_specs(128) + [_row_spec(128), _row_spec(1), _full_spec((1, 128))],
        [_row_spec(128)],
        [jax.ShapeDtypeStruct((N, 128), jnp.float32)],
        (parts, parts, g4, dinv, b4),
    )[0]


def kernel(features, edge_index, W1, b1, W2, b2, b3, b4, head1):
    src, dst = edge_index[0], edge_index[1]
    ppw = EPW - E // NW
    dsrc = jnp.zeros((NW, ppw), jnp.int32)
    ddst = jnp.broadcast_to(N + (jnp.arange(ppw, dtype=jnp.int32) % (N1 - N)),
                            (NW, ppw))
    srcp = jnp.concatenate(
        [src.reshape(NW, E // NW), dsrc], axis=1).reshape(NW, CH, 128)
    dstp = jnp.concatenate(
        [dst.reshape(NW, E // NW), ddst], axis=1).reshape(NW, CH, 128)

    zeros16 = jnp.zeros((N1, DEGW), jnp.float32)
    zeros64 = jnp.zeros((N1, 64), jnp.float32)
    zeros128 = jnp.zeros((N1, 128), jnp.float32)
    ones16 = jnp.zeros((128, DEGW), jnp.float32).at[:, 0].set(1.0)

    degp = _sc_degree()(dstp, ones16, zeros16)
    dinv, g1 = _tc_b(degp, features, W1.T)

    spmm128 = _sc_spmm(128)
    spmm64 = _sc_spmm_deep(64)

    p1 = spmm128(g1, srcp, dstp, zeros128)
    g2 = _tc_mid(p1, g1, dinv, b1.reshape(1, -1), W2.T, 128, 64)

    p2 = spmm64(g2, srcp, dstp, zeros64)
    h2, z, g3 = _tc_f(p2, g2, dinv, b2.reshape(1, -1), head1, W2)

    p3 = spmm128(g3, srcp, dstp, zeros128)
    g4 = _tc_mid(p3, g3, dinv, b3.reshape(1, -1), W1, 128, 128)

    p4 = spmm128(g4, srcp, dstp, zeros128)
    h4 = _tc_last(p4, g4, dinv, b4.reshape(1, -1))

    return (z, h2, h4)


# speedup vs baseline: 1.0006x; 1.0006x over previous
import functools

import jax
import jax.numpy as jnp
from jax import lax
from jax.experimental import pallas as pl
from jax.experimental.pallas import tpu as pltpu
from jax.experimental.pallas import tpu_sc as plsc

N = 10000
E = 320000
NW = 32
CH = 80
CHS = 40
EPW = CH * 128
EPAD = NW * EPW
N1 = 10112
RPT = N1 // 16
DEGW = 16

_MESH = plsc.VectorSubcoreMesh(core_axis_name="c", subcore_axis_name="s")


def _sc_degree():
    @functools.partial(
        pl.kernel,
        out_type=jax.ShapeDtypeStruct((2, N1, DEGW), jnp.float32),
        mesh=_MESH,
        scratch_types=[
            pltpu.VMEM((CH, 128), jnp.int32),
            pltpu.VMEM((128, DEGW), jnp.float32),
            pltpu.VMEM_SHARED((N1, DEGW), jnp.float32),
        ],
        compiler_params=pltpu.CompilerParams(use_tc_tiling_on_sc=False),
    )
    def k(dstp_hbm, ones_hbm, zeros_hbm, out_hbm, dst_v, ones_v, acc):
        c = lax.axis_index("c")
        s = lax.axis_index("s")
        wid = s * 2 + c
        row0 = s * RPT
        pltpu.sync_copy(zeros_hbm.at[pl.ds(row0, RPT)], acc.at[pl.ds(row0, RPT)])
        pltpu.sync_copy(dstp_hbm.at[wid], dst_v)
        pltpu.sync_copy(ones_hbm, ones_v)
        plsc.subcore_barrier()

        def body(j, carry):
            pltpu.sync_copy(ones_v, acc.at[dst_v.at[j]], add=True)
            return carry

        lax.fori_loop(0, CH, body, 0)
        plsc.subcore_barrier()
        pltpu.sync_copy(acc.at[pl.ds(row0, RPT)], out_hbm.at[c, pl.ds(row0, RPT)])

    return k


def _sc_spmm(D):
    @functools.partial(
        pl.kernel,
        out_type=jax.ShapeDtypeStruct((2, N1, D), jnp.float32),
        mesh=_MESH,
        scratch_types=[
            pltpu.VMEM((CHS, 128), jnp.int32),
            pltpu.VMEM((CHS, 128), jnp.int32),
            pltpu.VMEM((2, 128, D), jnp.float32),
            pltpu.VMEM_SHARED((N1, D), jnp.float32),
            pltpu.SemaphoreType.DMA,
            pltpu.SemaphoreType.DMA,
        ],
        compiler_params=pltpu.CompilerParams(use_tc_tiling_on_sc=False),
    )
    def k(g_hbm, srcp_hbm, dstp_hbm, zeros_hbm, out_hbm,
          src_v, dst_v, rows_v, acc, sem0, sem1):
        c = lax.axis_index("c")
        s = lax.axis_index("s")
        wid = s * 2 + c
        row0 = s * RPT
        pltpu.sync_copy(zeros_hbm.at[pl.ds(row0, RPT)], acc.at[pl.ds(row0, RPT)])
        plsc.subcore_barrier()

        sems = (sem0, sem1)
        for half in range(CH // CHS):
            pltpu.sync_copy(srcp_hbm.at[wid, pl.ds(half * CHS, CHS)], src_v)
            pltpu.sync_copy(dstp_hbm.at[wid, pl.ds(half * CHS, CHS)], dst_v)
            pltpu.async_copy(g_hbm.at[src_v.at[0]], rows_v.at[0], sem0)
            pltpu.async_copy(g_hbm.at[src_v.at[1]], rows_v.at[1], sem1)

            def body(i, carry):
                g0 = i * 2
                for b in range(2):
                    j = g0 + b
                    pltpu.make_async_copy(
                        g_hbm.at[src_v.at[j]], rows_v.at[b], sems[b]).wait()
                    pltpu.sync_copy(rows_v.at[b], acc.at[dst_v.at[j]], add=True)

                    @pl.when(j + 2 < CHS)
                    def _():
                        pltpu.async_copy(
                            g_hbm.at[src_v.at[j + 2]], rows_v.at[b], sems[b])
                return carry

            lax.fori_loop(0, CHS // 2, body, 0)
        plsc.subcore_barrier()
        pltpu.sync_copy(acc.at[pl.ds(row0, RPT)], out_hbm.at[c, pl.ds(row0, RPT)])

    return k


def _sc_spmm_deep(D, NBUF=4, G=2):
    @functools.partial(
        pl.kernel,
        out_type=jax.ShapeDtypeStruct((2, N1, D), jnp.float32),
        mesh=_MESH,
        scratch_types=[
            pltpu.VMEM((CH, 128), jnp.int32),
            pltpu.VMEM((CH, 128), jnp.int32),
            pltpu.VMEM((NBUF, 128, D), jnp.float32),
            pltpu.VMEM_SHARED((N1, D), jnp.float32),
            [pltpu.SemaphoreType.DMA] * NBUF,
            [pltpu.SemaphoreType.DMA] * NBUF,
        ],
        compiler_params=pltpu.CompilerParams(
            use_tc_tiling_on_sc=False) if D % 128 else None,
    )
    def k(g_hbm, srcp_hbm, dstp_hbm, zeros_hbm, out_hbm,
          src_v, dst_v, rows_v, acc, gsem, ssem):
        c = lax.axis_index("c")
        s = lax.axis_index("s")
        wid = s * 2 + c
        row0 = s * RPT
        pltpu.sync_copy(zeros_hbm.at[pl.ds(row0, RPT)], acc.at[pl.ds(row0, RPT)])
        pltpu.sync_copy(srcp_hbm.at[wid], src_v)
        pltpu.sync_copy(dstp_hbm.at[wid], dst_v)
        plsc.subcore_barrier()

        for j in range(G):
            pltpu.async_copy(g_hbm.at[src_v.at[j]], rows_v.at[j], gsem[j])

        def body(io, carry):
            j0 = io * NBUF
            for b in range(NBUF):
                j = j0 + b
                pltpu.make_async_copy(
                    g_hbm.at[src_v.at[j]], rows_v.at[b], gsem[b]).wait()
                pltpu.async_copy(rows_v.at[b], acc.at[dst_v.at[j]], ssem[b],
                                 add=True)
                k_next = j + G
                rk = (b + G) % NBUF

                @pl.when(k_next < CH)
                def _():
                    @pl.when(k_next >= NBUF)
                    def _():
                        pltpu.make_async_copy(
                            rows_v.at[rk],
                            acc.at[dst_v.at[jnp.maximum(k_next - NBUF, 0)]],
                            ssem[rk]).wait()

                    pltpu.async_copy(
                        g_hbm.at[src_v.at[k_next]], rows_v.at[rk], gsem[rk])
            return carry

        lax.fori_loop(0, CH // NBUF, body, 0)
        for d in range(NBUF):
            j = CH - NBUF + d
            pltpu.make_async_copy(
                rows_v.at[j % NBUF], acc.at[dst_v.at[j]],
                ssem[j % NBUF]).wait()
        plsc.subcore_barrier()
        pltpu.sync_copy(acc.at[pl.ds(row0, RPT)], out_hbm.at[c, pl.ds(row0, RPT)])

    return k


_RB = 1000


def _row_spec(D):
    return pl.BlockSpec((_RB, D), lambda i: (i, 0))


def _part_specs(D):
    return [pl.BlockSpec((1, _RB, D), lambda i, c=c: (c, i, 0)) for c in (0, 1)]


def _full_spec(shape):
    return pl.BlockSpec(shape, lambda i: tuple(0 for _ in shape))


def _tc_call(body, in_specs, out_specs, out_shapes, args):
    return pl.pallas_call(
        body,
        grid=(N // _RB,),
        in_specs=in_specs,
        out_specs=out_specs,
        out_shape=out_shapes,
    )(*args)


def _tc_b(degp, x, w1t):
    def body(p0, p1, x_r, w_r, dinv_r, g_r):
        deg = p0[0, :, 0:1] + p1[0, :, 0:1] + 1.0
        dinv = lax.rsqrt(deg)
        dinv_r[...] = dinv
        g_r[...] = dinv * jnp.dot(x_r[...], w_r[...],
                                  preferred_element_type=jnp.float32)

    return _tc_call(
        body,
        _part_specs(DEGW) + [_row_spec(128), _full_spec((128, 128))],
        [_row_spec(1), _row_spec(128)],
        [jax.ShapeDtypeStruct((N, 1), jnp.float32),
         jax.ShapeDtypeStruct((N, 128), jnp.float32)],
        (degp, degp, x, w1t),
    )


def _tc_mid(parts, g, dinv, b, wnext, D, DN):
    def body(p0, p1, g_r, dinv_r, b_r, w_r, gn_r):
        h = dinv_r[...] * (p0[0] + p1[0] + g_r[...]) + b_r[...]
        gn_r[...] = dinv_r[...] * jnp.dot(h, w_r[...],
                                          preferred_element_type=jnp.float32)

    return _tc_call(
        body,
        _part_specs(D) + [_row_spec(D), _row_spec(1), _full_spec((1, D)),
                          _full_spec((D, DN))],
        [_row_spec(DN)],
        [jax.ShapeDtypeStruct((N, DN), jnp.float32)],
        (parts, parts, g, dinv, b, wnext),
    )[0]


def _tc_f(parts, g2, dinv, b2, head1, w2):
    def body(p0, p1, g_r, dinv_r, b_r, h1_r, w2_r, h2_r, z_r, g3_r):
        h2 = dinv_r[...] * (p0[0] + p1[0] + g_r[...]) + b_r[...]
        z = jnp.dot(h2, h1_r[...], preferred_element_type=jnp.float32)
        h2_r[...] = h2
        z_r[...] = z
        g3_r[...] = dinv_r[...] * jnp.dot(z, w2_r[...],
                                          preferred_element_type=jnp.float32)

    return _tc_call(
        body,
        _part_specs(64) + [_row_spec(64), _row_spec(1), _full_spec((1, 64)),
                           _full_spec((64, 64)), _full_spec((64, 128))],
        [_row_spec(64), _row_spec(64), _row_spec(128)],
        [jax.ShapeDtypeStruct((N, 64), jnp.float32),
         jax.ShapeDtypeStruct((N, 64), jnp.float32),
         jax.ShapeDtypeStruct((N, 128), jnp.float32)],
        (parts, parts, g2, dinv, b2, head1, w2),
    )


def _tc_last(parts, g4, dinv, b4):
    def body(p0, p1, g_r, dinv_r, b_r, h_r):
        h_r[...] = dinv_r[...] * (p0[0] + p1[0] + g_r[...]) + b_r[...]

    return _tc_call(
        body,
        _part_specs(128) + [_row_spec(128), _row_spec(1), _full_spec((1, 128))],
        [_row_spec(128)],
        [jax.ShapeDtypeStruct((N, 128), jnp.float32)],
        (parts, parts, g4, dinv, b4),
    )[0]


def kernel(features, edge_index, W1, b1, W2, b2, b3, b4, head1):
    src, dst = edge_index[0], edge_index[1]
    ppw = EPW - E // NW
    dsrc = jnp.zeros((NW, ppw), jnp.int32)
    ddst = jnp.broadcast_to(N + (jnp.arange(ppw, dtype=jnp.int32) % (N1 - N)),
                            (NW, ppw))
    srcp = jnp.concatenate(
        [src.reshape(NW, E // NW), dsrc], axis=1).reshape(NW, CH, 128)
    dstp = jnp.concatenate(
        [dst.reshape(NW, E // NW), ddst], axis=1).reshape(NW, CH, 128)

    zeros16 = jnp.zeros((N1, DEGW), jnp.float32)
    zeros64 = jnp.zeros((N1, 64), jnp.float32)
    zeros128 = jnp.zeros((N1, 128), jnp.float32)
    ones16 = jnp.zeros((128, DEGW), jnp.float32).at[:, 0].set(1.0)

    degp = _sc_degree()(dstp, ones16, zeros16)
    dinv, g1 = _tc_b(degp, features, W1.T)

    spmm128 = _sc_spmm(128)
    spmm64 = _sc_spmm_deep(64)

    p1 = spmm128(g1, srcp, dstp, zeros128)
    g2 = _tc_mid(p1, g1, dinv, b1.reshape(1, -1), W2.T, 128, 64)

    p2 = spmm64(g2, srcp, dstp, zeros64)
    h2, z, g3 = _tc_f(p2, g2, dinv, b2.reshape(1, -1), head1, W2)

    p3 = spmm128(g3, srcp, dstp, zeros128)
    g4 = _tc_mid(p3, g3, dinv, b3.reshape(1, -1), W1, 128, 128)

    p4 = spmm128(g4, srcp, dstp, zeros128)
    h4 = _tc_last(p4, g4, dinv, b4.reshape(1, -1))

    return (z, h2, h4)
